# Initial kernel scaffold; baseline (speedup 1.0000x reference)
import jax
import jax.numpy as jnp
from jax.experimental import pallas as pl


def kernel(xyz_q, xyz_k, v_k):
    raise NotImplementedError("write your pallas kernel here")


# speedup vs baseline: 30.5260x; 30.5260x over previous
import functools

import jax
import jax.numpy as jnp
from jax import lax
from jax.experimental import pallas as pl
from jax.experimental.pallas import tpu as pltpu
from jax.experimental.pallas import tpu_sc as plsc

B = 4
NQ = 8192
NK = 2048
C = 32
TQ = 512
NW = 32
QPW = B * NQ // NW
L = 16


def _tc_topk_body(q_ref, k_ref, i1_ref, i2_ref, i3_ref, w1_ref, w2_ref, w3_ref):
    q = q_ref[0]
    kt = k_ref[0]
    qq = jnp.sum(q * q, axis=1, keepdims=True)
    kk = jnp.sum(kt * kt, axis=0, keepdims=True)
    d = -2.0 * jnp.dot(q, kt, preferred_element_type=jnp.float32) + qq + kk

    iota = lax.broadcasted_iota(jnp.int32, (TQ, NK), 1)
    inf = jnp.float32(jnp.inf)

    m1 = jnp.min(d, axis=1, keepdims=True)
    i1 = jnp.min(jnp.where(d == m1, iota, NK), axis=1, keepdims=True)
    d = jnp.where(iota == i1, inf, d)
    m2 = jnp.min(d, axis=1, keepdims=True)
    i2 = jnp.min(jnp.where(d == m2, iota, NK), axis=1, keepdims=True)
    d = jnp.where(iota == i2, inf, d)
    m3 = jnp.min(d, axis=1, keepdims=True)
    i3 = jnp.min(jnp.where(d == m3, iota, NK), axis=1, keepdims=True)

    w1 = 1.0 / jnp.maximum(m1, 1e-10)
    w2 = 1.0 / jnp.maximum(m2, 1e-10)
    w3 = 1.0 / jnp.maximum(m3, 1e-10)
    s = w1 + w2 + w3
    i1_ref[0] = i1
    i2_ref[0] = i2
    i3_ref[0] = i3
    w1_ref[0] = w1 / s
    w2_ref[0] = w2 / s
    w3_ref[0] = w3 / s


def _tc_topk(q8, k8t, interpret=False):
    grid = (B, NQ // TQ)
    out = pl.pallas_call(
        _tc_topk_body,
        grid=grid,
        in_specs=[
            pl.BlockSpec((1, TQ, 8), lambda b, i: (b, i, 0)),
            pl.BlockSpec((1, 8, NK), lambda b, i: (b, 0, 0)),
        ],
        out_specs=[pl.BlockSpec((1, TQ, 1), lambda b, i: (b, i, 0))] * 6,
        out_shape=[
            jax.ShapeDtypeStruct((B, NQ, 1), jnp.int32),
            jax.ShapeDtypeStruct((B, NQ, 1), jnp.int32),
            jax.ShapeDtypeStruct((B, NQ, 1), jnp.int32),
            jax.ShapeDtypeStruct((B, NQ, 1), jnp.float32),
            jax.ShapeDtypeStruct((B, NQ, 1), jnp.float32),
            jax.ShapeDtypeStruct((B, NQ, 1), jnp.float32),
        ],
        compiler_params=pltpu.CompilerParams(
            dimension_semantics=("parallel", "parallel"),
        ),
        interpret=interpret,
    )(q8, k8t)
    return out


def _sc_body(i1, i2, i3, w1, w2, w3, vk, out,
             vk_v, i1_v, i2_v, i3_v, w1_v, w2_v, w3_v, out_v):
    cidx = lax.axis_index("c")
    sidx = lax.axis_index("s")
    wid = sidx * 2 + cidx
    b = wid // (NW // B)
    base = wid * QPW

    pltpu.sync_copy(vk.at[b], vk_v)
    pltpu.sync_copy(i1.at[pl.ds(base, QPW)], i1_v)
    pltpu.sync_copy(i2.at[pl.ds(base, QPW)], i2_v)
    pltpu.sync_copy(i3.at[pl.ds(base, QPW)], i3_v)
    pltpu.sync_copy(w1.at[pl.ds(base, QPW)], w1_v)
    pltpu.sync_copy(w2.at[pl.ds(base, QPW)], w2_v)
    pltpu.sync_copy(w3.at[pl.ds(base, QPW)], w3_v)

    def group(g, carry):
        q0 = g * L
        qv = (q0 + lax.iota(jnp.int32, L)) * C
        ia = i1_v[pl.ds(q0, L)] * C
        ib = i2_v[pl.ds(q0, L)] * C
        ic = i3_v[pl.ds(q0, L)] * C
        wa = w1_v[pl.ds(q0, L)]
        wb = w2_v[pl.ds(q0, L)]
        wc = w3_v[pl.ds(q0, L)]
        for ch in range(C):
            va = plsc.load_gather(vk_v, [ia + ch])
            vb = plsc.load_gather(vk_v, [ib + ch])
            vc = plsc.load_gather(vk_v, [ic + ch])
            acc = wa * va + wb * vb + wc * vc
            plsc.store_scatter(out_v, [qv + ch], acc)
        return carry

    lax.fori_loop(0, QPW // L, group, 0)
    pltpu.sync_copy(out_v, out.at[pl.ds(base * C, QPW * C)])


@functools.cache
def _sc_combine_fn():
    return functools.partial(
        pl.kernel,
        mesh=plsc.VectorSubcoreMesh(core_axis_name="c", subcore_axis_name="s"),
        out_type=jax.ShapeDtypeStruct((B * NQ * C,), jnp.float32),
        scratch_types=[
            pltpu.VMEM((NK * C,), jnp.float32),
            pltpu.VMEM((QPW,), jnp.int32),
            pltpu.VMEM((QPW,), jnp.int32),
            pltpu.VMEM((QPW,), jnp.int32),
            pltpu.VMEM((QPW,), jnp.float32),
            pltpu.VMEM((QPW,), jnp.float32),
            pltpu.VMEM((QPW,), jnp.float32),
            pltpu.VMEM((QPW * C,), jnp.float32),
        ],
        compiler_params=pltpu.CompilerParams(needs_layout_passes=False),
    )(_sc_body)


def _pad8(xyz):
    b, n, _ = xyz.shape
    return jnp.concatenate(
        [xyz, jnp.zeros((b, n, 5), dtype=xyz.dtype)], axis=-1)


@jax.jit
def kernel(xyz_q, xyz_k, v_k):
    q8 = _pad8(xyz_q)
    k8t = jnp.swapaxes(_pad8(xyz_k), 1, 2)
    i1, i2, i3, w1, w2, w3 = _tc_topk(q8, k8t)
    flat = lambda x: x.reshape(B * NQ)
    out = _sc_combine_fn()(flat(i1), flat(i2), flat(i3),
                           flat(w1), flat(w2), flat(w3),
                           v_k.reshape(B, NK * C))
    return out.reshape(B, NQ, C)


# speedup vs baseline: 49.9253x; 1.6355x over previous
import functools

import jax
import jax.numpy as jnp
from jax import lax
from jax.experimental import pallas as pl
from jax.experimental.pallas import tpu as pltpu
from jax.experimental.pallas import tpu_sc as plsc

B = 4
NQ = 8192
NK = 2048
C = 32
CP = 33
TQ = 512
NS = 16
NW = 32
QPW = B * NQ // NW
L = 16


def _tc_topk_body(q_ref, k_ref, i1_ref, i2_ref, i3_ref, w1_ref, w2_ref, w3_ref):
    qs = q_ref[0]
    kt = k_ref[0]
    qq = 0.25 * jnp.sum(qs * qs, axis=1, keepdims=True)
    kk = jnp.sum(kt * kt, axis=0, keepdims=True)
    d = jnp.dot(qs, kt, preferred_element_type=jnp.float32) + qq + kk
    d = jnp.maximum(d, 1e-10)
    bits = lax.bitcast_convert_type(d, jnp.int32)

    mask = jnp.int32(-16)
    big = jnp.full((TQ, 128), jnp.float32(jnp.inf))
    a1, a2, a3 = big, big, big
    for s in range(NS):
        x = lax.bitcast_convert_type(
            (bits[:, s * 128:(s + 1) * 128] & mask) | s, jnp.float32)
        hi = jnp.maximum(a1, x)
        a1 = jnp.minimum(a1, x)
        hi2 = jnp.maximum(a2, hi)
        a2 = jnp.minimum(a2, hi)
        a3 = jnp.minimum(a3, hi2)

    lane = lax.broadcasted_iota(jnp.int32, (TQ, 128), 1).astype(jnp.float32)
    h, nxt = a1, a2
    keys, lanes = [], []
    for _ in range(3):
        m = jnp.min(h, axis=1, keepdims=True)
        l = jnp.min(jnp.where(h == m, lane, 128.0), axis=1, keepdims=True)
        keys.append(lax.bitcast_convert_type(m, jnp.int32))
        lanes.append(l)
        hit = lane == l
        h = jnp.where(hit, nxt, h)
        nxt = jnp.where(hit, a3, nxt)

    cols = [(k & 15) * 128 + l.astype(jnp.int32)
            for k, l in zip(keys, lanes)]
    v1 = lax.bitcast_convert_type(keys[0] & mask, jnp.float32)
    v2 = lax.bitcast_convert_type(keys[1] & mask, jnp.float32)
    v3 = lax.bitcast_convert_type(keys[2] & mask, jnp.float32)
    w1 = 1.0 / v1
    w2 = 1.0 / v2
    w3 = 1.0 / v3
    s = w1 + w2 + w3
    i1_ref[0] = cols[0]
    i2_ref[0] = cols[1]
    i3_ref[0] = cols[2]
    w1_ref[0] = w1 / s
    w2_ref[0] = w2 / s
    w3_ref[0] = w3 / s


def _tc_topk(q8, k8t, interpret=False):
    grid = (B, NQ // TQ)
    out = pl.pallas_call(
        _tc_topk_body,
        grid=grid,
        in_specs=[
            pl.BlockSpec((1, TQ, 8), lambda b, i: (b, i, 0)),
            pl.BlockSpec((1, 8, NK), lambda b, i: (b, 0, 0)),
        ],
        out_specs=[pl.BlockSpec((1, TQ, 1), lambda b, i: (b, i, 0))] * 6,
        out_shape=[
            jax.ShapeDtypeStruct((B, NQ, 1), jnp.int32),
            jax.ShapeDtypeStruct((B, NQ, 1), jnp.int32),
            jax.ShapeDtypeStruct((B, NQ, 1), jnp.int32),
            jax.ShapeDtypeStruct((B, NQ, 1), jnp.float32),
            jax.ShapeDtypeStruct((B, NQ, 1), jnp.float32),
            jax.ShapeDtypeStruct((B, NQ, 1), jnp.float32),
        ],
        compiler_params=pltpu.CompilerParams(
            dimension_semantics=("parallel", "parallel"),
        ),
        interpret=interpret,
    )(q8, k8t)
    return out


def _sc_body(i1, i2, i3, w1, w2, w3, vk, out,
             vk_v, i1_v, i2_v, i3_v, w1_v, w2_v, w3_v, out_v):
    cidx = lax.axis_index("c")
    sidx = lax.axis_index("s")
    wid = sidx * 2 + cidx
    b = wid // (NW // B)
    base = wid * QPW

    pltpu.sync_copy(vk.at[b], vk_v)
    pltpu.sync_copy(i1.at[pl.ds(base, QPW)], i1_v)
    pltpu.sync_copy(i2.at[pl.ds(base, QPW)], i2_v)
    pltpu.sync_copy(i3.at[pl.ds(base, QPW)], i3_v)
    pltpu.sync_copy(w1.at[pl.ds(base, QPW)], w1_v)
    pltpu.sync_copy(w2.at[pl.ds(base, QPW)], w2_v)
    pltpu.sync_copy(w3.at[pl.ds(base, QPW)], w3_v)

    def group(g, carry):
        q0 = g * L
        ia = i1_v[pl.ds(q0, L)] * CP
        ib = i2_v[pl.ds(q0, L)] * CP
        ic = i3_v[pl.ds(q0, L)] * CP
        wa = w1_v[pl.ds(q0, L)]
        wb = w2_v[pl.ds(q0, L)]
        wc = w3_v[pl.ds(q0, L)]
        for ch in range(C):
            va = plsc.load_gather(vk_v, [ia + ch])
            vb = plsc.load_gather(vk_v, [ib + ch])
            vc = plsc.load_gather(vk_v, [ic + ch])
            acc = wa * va + wb * vb + wc * vc
            out_v[pl.ds(ch * QPW + q0, L)] = acc
        return carry

    lax.fori_loop(0, QPW // L, group, 0)
    pltpu.sync_copy(out_v, out.at[pl.ds(wid * QPW * C, QPW * C)])


@functools.cache
def _sc_combine_fn():
    return functools.partial(
        pl.kernel,
        mesh=plsc.VectorSubcoreMesh(core_axis_name="c", subcore_axis_name="s"),
        out_type=jax.ShapeDtypeStruct((NW * C * QPW,), jnp.float32),
        scratch_types=[
            pltpu.VMEM((NK * CP,), jnp.float32),
            pltpu.VMEM((QPW,), jnp.int32),
            pltpu.VMEM((QPW,), jnp.int32),
            pltpu.VMEM((QPW,), jnp.int32),
            pltpu.VMEM((QPW,), jnp.float32),
            pltpu.VMEM((QPW,), jnp.float32),
            pltpu.VMEM((QPW,), jnp.float32),
            pltpu.VMEM((C * QPW,), jnp.float32),
        ],
        compiler_params=pltpu.CompilerParams(needs_layout_passes=False),
    )(_sc_body)


def _pad8(xyz, scale=1.0):
    b, n, _ = xyz.shape
    return jnp.concatenate(
        [scale * xyz, jnp.zeros((b, n, 5), dtype=xyz.dtype)], axis=-1)


@jax.jit
def kernel(xyz_q, xyz_k, v_k):
    q8 = _pad8(xyz_q, -2.0)
    k8t = jnp.swapaxes(_pad8(xyz_k), 1, 2)
    i1, i2, i3, w1, w2, w3 = _tc_topk(q8, k8t)
    vkp = jnp.concatenate(
        [v_k, jnp.zeros((B, NK, CP - C), jnp.float32)], axis=-1)
    flat = lambda x: x.reshape(B * NQ)
    out = _sc_combine_fn()(flat(i1), flat(i2), flat(i3),
                           flat(w1), flat(w2), flat(w3),
                           vkp.reshape(B, NK * CP))
    out = out.reshape(B, NW // B, C, QPW).transpose(0, 1, 3, 2)
    return out.reshape(B, NQ, C)
